# speedup vs baseline: 2.0404x; 2.0404x over previous
import jax
import jax.numpy as jnp
from jax.experimental import pallas as pl

VOCAB = 32768
HIDDEN = 2048
NUM_EXPERTS = 16
TOP_K = 2
TOKENS = 512
EXPERT_VOCAB = VOCAB // NUM_EXPERTS


def _moe_head_body(x_ref, w_ref, rw_ref, out_ref):
    x = x_ref[...]
    rw = rw_ref[...]
    logits = jnp.dot(x, rw.T, preferred_element_type=jnp.float32)
    m = jnp.max(logits, axis=1, keepdims=True)
    w = jnp.exp(logits - m)
    w = w / jnp.sum(w, axis=1, keepdims=True)
    a1 = jnp.argmax(w, axis=1)
    eids = jax.lax.broadcasted_iota(jnp.int32, (TOKENS, NUM_EXPERTS), 1)
    w2 = jnp.where(eids == a1[:, None], -jnp.inf, w)
    a2 = jnp.argmax(w2, axis=1)
    e = pl.program_id(0)
    sel = (a1 == e) | (a2 == e)
    acc = jnp.dot(x, w_ref[0].T, preferred_element_type=jnp.float32)
    out_ref[...] = jnp.where(sel[:, None], acc, -jnp.inf)


def kernel(hidden_states, expert_weight, router_weight):
    return pl.pallas_call(
        _moe_head_body,
        grid=(NUM_EXPERTS,),
        in_specs=[
            pl.BlockSpec((TOKENS, HIDDEN), lambda e: (0, 0)),
            pl.BlockSpec((1, EXPERT_VOCAB, HIDDEN), lambda e: (e, 0, 0)),
            pl.BlockSpec((NUM_EXPERTS, HIDDEN), lambda e: (0, 0)),
        ],
        out_specs=pl.BlockSpec((TOKENS, EXPERT_VOCAB), lambda e: (0, e)),
        out_shape=jax.ShapeDtypeStruct((TOKENS, VOCAB), jnp.float32),
    )(hidden_states, expert_weight, router_weight)


# speedup vs baseline: 2.0416x; 1.0006x over previous
import jax
import jax.numpy as jnp
from jax.experimental import pallas as pl

VOCAB = 32768
HIDDEN = 2048
NUM_EXPERTS = 16
TOP_K = 2
TOKENS = 512
EXPERT_VOCAB = VOCAB // NUM_EXPERTS


def _moe_head_body(x_ref, w_ref, rw_ref, out_ref):
    x = x_ref[...]
    rw = rw_ref[...]
    logits = jnp.dot(x, rw.T, preferred_element_type=jnp.float32)
    m = jnp.max(logits, axis=1, keepdims=True)
    w = jnp.exp(logits - m)
    w = w / jnp.sum(w, axis=1, keepdims=True)
    a1 = jnp.argmax(w, axis=1)
    eids = jax.lax.broadcasted_iota(jnp.int32, (TOKENS, NUM_EXPERTS), 1)
    w2 = jnp.where(eids == a1[:, None], -jnp.inf, w)
    a2 = jnp.argmax(w2, axis=1)
    e = pl.program_id(0)
    sel = (a1 == e) | (a2 == e)
    xb = x.astype(jnp.bfloat16)
    wb = w_ref[0].astype(jnp.bfloat16)
    acc = jnp.dot(xb, wb.T, preferred_element_type=jnp.float32)
    out_ref[...] = jnp.where(sel[:, None], acc, -jnp.inf)


def kernel(hidden_states, expert_weight, router_weight):
    return pl.pallas_call(
        _moe_head_body,
        grid=(NUM_EXPERTS,),
        in_specs=[
            pl.BlockSpec((TOKENS, HIDDEN), lambda e: (0, 0)),
            pl.BlockSpec((1, EXPERT_VOCAB, HIDDEN), lambda e: (e, 0, 0)),
            pl.BlockSpec((NUM_EXPERTS, HIDDEN), lambda e: (0, 0)),
        ],
        out_specs=pl.BlockSpec((TOKENS, EXPERT_VOCAB), lambda e: (0, e)),
        out_shape=jax.ShapeDtypeStruct((TOKENS, VOCAB), jnp.float32),
    )(hidden_states, expert_weight, router_weight)
